# Initial kernel scaffold; baseline (speedup 1.0000x reference)
import jax
import jax.numpy as jnp
from jax.experimental import pallas as pl


def kernel(x, sub_representations, new_nodes):
    raise NotImplementedError("write your pallas kernel here")


# speedup vs baseline: 1.1494x; 1.1494x over previous
import functools

import jax
import jax.numpy as jnp
from jax import lax
from jax.experimental import pallas as pl
from jax.experimental.pallas import tpu as pltpu
from jax.experimental.pallas import tpu_sc as plsc

S = 50000
D = 128
C = 200
NCHUNK = S // C
NC, NS = 2, 16
NW = NC * NS
CPW = (NCHUNK + NW - 1) // NW


def _sc_body(x_hbm, sub_hbm, idx_hbm, out_hbm, idx_v, rows_v, sub_v, sem):
    wid = lax.axis_index("s") * NC + lax.axis_index("c")

    def chunk_body(i, carry):
        chunk = wid + i * NW

        @pl.when(chunk < NCHUNK)
        def _():
            base = chunk * C
            pltpu.sync_copy(idx_hbm.at[pl.ds(base, C)], idx_v)
            g = pltpu.async_copy(x_hbm.at[idx_v], rows_v, sem)
            s = pltpu.async_copy(sub_hbm.at[pl.ds(base, C)], sub_v, sem)
            g.wait()
            s.wait()

            def add_row(r, c2):
                for c8 in range(D // 16):
                    sl = pl.ds(c8 * 16, 16)
                    rows_v[r, sl] = rows_v[r, sl] + sub_v[r, sl]
                return c2

            lax.fori_loop(0, C, add_row, 0)
            pltpu.sync_copy(rows_v, out_hbm.at[pl.ds(base, C)])

        return carry

    lax.fori_loop(0, CPW, chunk_body, 0)


def kernel(x, sub_representations, new_nodes):
    idx = new_nodes.reshape(-1)

    mesh = plsc.VectorSubcoreMesh(core_axis_name="c", subcore_axis_name="s")
    run = functools.partial(
        pl.kernel,
        mesh=mesh,
        out_type=jax.ShapeDtypeStruct((S, D), jnp.float32),
        scratch_types=[
            pltpu.VMEM((C,), jnp.int32),
            pltpu.VMEM((C, D), jnp.float32),
            pltpu.VMEM((C, D), jnp.float32),
            pltpu.SemaphoreType.DMA,
        ],
    )(_sc_body)
    return run(x, sub_representations, idx)


# speedup vs baseline: 1.3233x; 1.1513x over previous
import functools

import jax
import jax.numpy as jnp
from jax import lax
from jax.experimental import pallas as pl
from jax.experimental.pallas import tpu as pltpu
from jax.experimental.pallas import tpu_sc as plsc

S = 50000
D = 128
C = 200
NCHUNK = S // C
NC, NS = 2, 16
NW = NC * NS
CPW = (NCHUNK + NW - 1) // NW


def _sc_body(x_hbm, sub_hbm, idx_hbm, out_hbm, idx_v, rows_v, sub_v, sem):
    wid = lax.axis_index("s") * NC + lax.axis_index("c")

    def chunk_body(i, carry):
        chunk = wid + i * NW

        @pl.when(chunk < NCHUNK)
        def _():
            base = chunk * C
            pltpu.sync_copy(idx_hbm.at[pl.ds(base, C)], idx_v)
            pltpu.sync_copy(sub_hbm.at[pl.ds(base, C)], rows_v)
            pltpu.async_copy(x_hbm.at[idx_v], rows_v, sem, add=True).wait()
            pltpu.sync_copy(rows_v, out_hbm.at[pl.ds(base, C)])

        return carry

    lax.fori_loop(0, CPW, chunk_body, 0)


def kernel(x, sub_representations, new_nodes):
    idx = new_nodes.reshape(-1)

    mesh = plsc.VectorSubcoreMesh(core_axis_name="c", subcore_axis_name="s")
    run = functools.partial(
        pl.kernel,
        mesh=mesh,
        out_type=jax.ShapeDtypeStruct((S, D), jnp.float32),
        scratch_types=[
            pltpu.VMEM((C,), jnp.int32),
            pltpu.VMEM((C, D), jnp.float32),
            pltpu.VMEM((C, D), jnp.float32),
            pltpu.SemaphoreType.DMA,
        ],
    )(_sc_body)
    return run(x, sub_representations, idx)


# speedup vs baseline: 1.5723x; 1.1882x over previous
import functools

import jax
import jax.numpy as jnp
from jax import lax
from jax.experimental import pallas as pl
from jax.experimental.pallas import tpu as pltpu
from jax.experimental.pallas import tpu_sc as plsc

S = 50000
D = 128
C = 200
NCHUNK = S // C
NC, NS = 2, 16
NW = NC * NS
NU = NCHUNK // NW
CPW = (NCHUNK + NW - 1) // NW
NB = 3


def _sc_body(x_hbm, sub_hbm, idx_hbm, out_hbm,
             idx0, idx1, idx2, rows0, rows1, rows2,
             si0, si1, si2, ss0, ss1, ss2, sg0, sg1, sg2, so0, so1, so2):
    wid = lax.axis_index("s") * NC + lax.axis_index("c")
    idx_b = (idx0, idx1, idx2)
    rows = (rows0, rows1, rows2)
    sem_i = (si0, si1, si2)
    sem_s = (ss0, ss1, ss2)
    sem_g = (sg0, sg1, sg2)
    sem_o = (so0, so1, so2)

    def issue_loads(i):
        b = i % NB
        base = (wid + i * NW) * C
        pltpu.async_copy(idx_hbm.at[pl.ds(base, C)], idx_b[b], sem_i[b])
        pltpu.async_copy(sub_hbm.at[pl.ds(base, C)], rows[b], sem_s[b])

    def issue_gadd(i):
        b = i % NB
        pltpu.make_async_copy(idx_hbm.at[pl.ds(0, C)], idx_b[b], sem_i[b]).wait()
        pltpu.make_async_copy(sub_hbm.at[pl.ds(0, C)], rows[b], sem_s[b]).wait()
        pltpu.async_copy(x_hbm.at[idx_b[b]], rows[b], sem_g[b], add=True)

    def issue_store(i):
        b = i % NB
        pltpu.make_async_copy(x_hbm.at[idx_b[b]], rows[b], sem_g[b]).wait()
        base = (wid + i * NW) * C
        pltpu.async_copy(rows[b], out_hbm.at[pl.ds(base, C)], sem_o[b])

    def drain_store(i):
        b = i % NB
        pltpu.make_async_copy(rows[b], out_hbm.at[pl.ds(0, C)], sem_o[b]).wait()

    issue_loads(0)
    if NU > 1:
        issue_loads(1)
    issue_gadd(0)
    drained = -1
    for i in range(NU):
        if i + 2 < NU:
            if i + 2 - NB >= 0:
                drain_store(i + 2 - NB)
                drained = i + 2 - NB
            issue_loads(i + 2)
        issue_store(i)
        if i + 1 < NU:
            issue_gadd(i + 1)
    for i in range(drained + 1, NU):
        drain_store(i)

    for i in range(NU, CPW):
        chunk = wid + i * NW

        @pl.when(chunk < NCHUNK)
        def _():
            base = chunk * C
            pltpu.sync_copy(idx_hbm.at[pl.ds(base, C)], idx_b[0])
            pltpu.sync_copy(sub_hbm.at[pl.ds(base, C)], rows[0])
            pltpu.async_copy(x_hbm.at[idx_b[0]], rows[0], sem_g[0],
                             add=True).wait()
            pltpu.sync_copy(rows[0], out_hbm.at[pl.ds(base, C)])


def kernel(x, sub_representations, new_nodes):
    idx = new_nodes.reshape(-1)

    mesh = plsc.VectorSubcoreMesh(core_axis_name="c", subcore_axis_name="s")
    run = functools.partial(
        pl.kernel,
        mesh=mesh,
        out_type=jax.ShapeDtypeStruct((S, D), jnp.float32),
        scratch_types=(
            [pltpu.VMEM((C,), jnp.int32) for _ in range(NB)]
            + [pltpu.VMEM((C, D), jnp.float32) for _ in range(NB)]
            + [pltpu.SemaphoreType.DMA for _ in range(4 * NB)]
        ),
    )(_sc_body)
    return run(x, sub_representations, idx)


# speedup vs baseline: 1.6503x; 1.0496x over previous
import functools

import jax
import jax.numpy as jnp
from jax import lax
from jax.experimental import pallas as pl
from jax.experimental.pallas import tpu as pltpu
from jax.experimental.pallas import tpu_sc as plsc

S = 50000
D = 128
NC, NS = 2, 16
NW = NC * NS
NB = 4
CMAX = 224
SPAN_A, SPAN_B = 1568, 1480
NWA = 30
SIZES_A = [224] * 7
SIZES_B = [224] * 6 + [136]


def _span_pipeline(base, sizes, x_hbm, sub_hbm, idx_hbm, out_hbm,
                   idx_all, rows, sem_i, sem_s, sem_g, sem_o):
    K = len(sizes)
    offs = [sum(sizes[:j]) for j in range(K)]
    total = sum(sizes)

    def idx_desc():
        return pltpu.make_async_copy(
            idx_hbm.at[pl.ds(0, total)], idx_all.at[pl.ds(0, total)], sem_i)

    def L(j):
        b = j % NB
        pltpu.async_copy(sub_hbm.at[pl.ds(base + offs[j], sizes[j])],
                         rows[b].at[pl.ds(0, sizes[j])], sem_s[b])

    def G(j):
        b = j % NB
        pltpu.make_async_copy(sub_hbm.at[pl.ds(0, sizes[j])],
                              rows[b].at[pl.ds(0, sizes[j])], sem_s[b]).wait()
        pltpu.async_copy(x_hbm.at[idx_all.at[pl.ds(offs[j], sizes[j])]],
                         rows[b].at[pl.ds(0, sizes[j])], sem_g[b], add=True)

    def W(j):
        b = j % NB
        pltpu.make_async_copy(x_hbm.at[idx_all.at[pl.ds(offs[j], sizes[j])]],
                              rows[b].at[pl.ds(0, sizes[j])], sem_g[b]).wait()
        pltpu.async_copy(rows[b].at[pl.ds(0, sizes[j])],
                         out_hbm.at[pl.ds(base + offs[j], sizes[j])], sem_o[b])

    def Dr(j):
        b = j % NB
        pltpu.make_async_copy(rows[b].at[pl.ds(0, sizes[j])],
                              out_hbm.at[pl.ds(0, sizes[j])], sem_o[b]).wait()

    pltpu.async_copy(idx_hbm.at[pl.ds(base, total)],
                     idx_all.at[pl.ds(0, total)], sem_i)
    L(0)
    if K > 1:
        L(1)
    idx_desc().wait()
    G(0)
    drained = -1
    for j in range(K):
        if j + 2 < K:
            if j + 2 - NB >= 0:
                Dr(j + 2 - NB)
                drained = j + 2 - NB
            L(j + 2)
        W(j)
        if j + 1 < K:
            G(j + 1)
    for j in range(drained + 1, K):
        Dr(j)


def _sc_body(x_hbm, sub_hbm, idx_hbm, out_hbm,
             idx_all, rows0, rows1, rows2, rows3,
             si, ss0, ss1, ss2, ss3, sg0, sg1, sg2, sg3, so0, so1, so2, so3):
    wid = lax.axis_index("s") * NC + lax.axis_index("c")
    rows = (rows0, rows1, rows2, rows3)
    sem_s = (ss0, ss1, ss2, ss3)
    sem_g = (sg0, sg1, sg2, sg3)
    sem_o = (so0, so1, so2, so3)
    args = (x_hbm, sub_hbm, idx_hbm, out_hbm,
            idx_all, rows, si, sem_s, sem_g, sem_o)

    @pl.when(wid < NWA)
    def _():
        _span_pipeline(wid * SPAN_A, SIZES_A, *args)

    @pl.when(wid >= NWA)
    def _():
        _span_pipeline(NWA * SPAN_A + (wid - NWA) * SPAN_B, SIZES_B, *args)


def kernel(x, sub_representations, new_nodes):
    idx = new_nodes.reshape(-1)

    mesh = plsc.VectorSubcoreMesh(core_axis_name="c", subcore_axis_name="s")
    run = functools.partial(
        pl.kernel,
        mesh=mesh,
        out_type=jax.ShapeDtypeStruct((S, D), jnp.float32),
        scratch_types=(
            [pltpu.VMEM((SPAN_A,), jnp.int32)]
            + [pltpu.VMEM((CMAX, D), jnp.float32) for _ in range(NB)]
            + [pltpu.SemaphoreType.DMA for _ in range(1 + 3 * NB)]
        ),
    )(_sc_body)
    return run(x, sub_representations, idx)
